# Initial kernel scaffold; baseline (speedup 1.0000x reference)
import jax
import jax.numpy as jnp
from jax.experimental import pallas as pl


def kernel(incoming):
    raise NotImplementedError("write your pallas kernel here")


# speedup vs baseline: 3.0007x; 3.0007x over previous
import functools

import jax
import jax.numpy as jnp
from jax import lax
from jax.experimental import pallas as pl
from jax.experimental.pallas import tpu as pltpu
from jax.experimental.pallas import tpu_sc as plsc

_LANES = 16


@functools.lru_cache(maxsize=None)
def _make_kernel(B, L, N):
    info = plsc.get_sparse_core_info()
    NC, NS = info.num_cores, info.num_subcores
    NW = NC * NS
    assert (B * L) % NW == 0
    slabs_per_tile = (B * L) // NW
    assert L % slabs_per_tile == 0 or slabs_per_tile % L == 0
    KN = N // _LANES

    mesh = plsc.VectorSubcoreMesh(core_axis_name="c", subcore_axis_name="s")

    @functools.partial(
        pl.kernel,
        out_type=jax.ShapeDtypeStruct((B, L, L, 3 * N), jnp.float32),
        mesh=mesh,
        scratch_types=[
            pltpu.VMEM((L, N), jnp.float32),
            pltpu.VMEM((L, 3 * N), jnp.float32),
            pltpu.VMEM((L, 3 * N), jnp.float32),
            pltpu.SemaphoreType.DMA,
            pltpu.SemaphoreType.DMA,
        ],
    )
    def run(inc_hbm, out_hbm, table_v, slab0, slab1, sem0, sem1):
        wid = lax.axis_index("s") * NC + lax.axis_index("c")
        tpb = L // slabs_per_tile
        b = wid // tpb
        r0 = (wid % tpb) * slabs_per_tile

        pltpu.sync_copy(inc_hbm.at[b], table_v)

        slabs = (slab0, slab1)
        sems = (sem0, sem1)

        def init_body(c, carry):
            for k in range(KN):
                v = table_v[c, pl.ds(k * _LANES, _LANES)]
                slab0[c, pl.ds(k * _LANES, _LANES)] = v
                slab1[c, pl.ds(k * _LANES, _LANES)] = v
            return carry

        lax.fori_loop(0, L, init_body, 0)

        def build(slab, r):
            row = [table_v[r, pl.ds(k * _LANES, _LANES)] for k in range(KN)]

            def c_body(c, carry):
                m = (r + c) // 2
                for k in range(KN):
                    slab[c, pl.ds(N + k * _LANES, _LANES)] = row[k]
                    slab[c, pl.ds(2 * N + k * _LANES, _LANES)] = (
                        table_v[m, pl.ds(k * _LANES, _LANES)]
                    )
                return carry

            lax.fori_loop(0, L, c_body, 0)

        def outer(g, carry):
            for buf in range(2):
                r = r0 + g * 2 + buf

                @pl.when(g > 0)
                def _wait():
                    pltpu.make_async_copy(
                        slabs[buf], out_hbm.at[b, r0], sems[buf]
                    ).wait()

                build(slabs[buf], r)
                pltpu.async_copy(slabs[buf], out_hbm.at[b, r], sems[buf])
            return carry

        lax.fori_loop(0, slabs_per_tile // 2, outer, 0)

        for buf in range(2):
            pltpu.make_async_copy(
                slabs[buf], out_hbm.at[b, r0], sems[buf]
            ).wait()

    return run


def kernel(incoming):
    B, L, N = incoming.shape
    return _make_kernel(B, L, N)(incoming)
